# baseline (speedup 1.0000x reference)
import functools

import jax
import jax.numpy as jnp
from jax import lax
from jax.experimental import pallas as pl
from jax.experimental.pallas import tpu as pltpu
from jax.experimental.pallas import tpu_sc as plsc

N = 50000
E = 800000
HID = 32
HEADS = 4
DH = 8
BN = 400
NB = N // BN
C = 80
NSUB = 16
EPT = E // NSUB
NCH = EPT // C
RC = 200
NRC = N // RC
KMAX = (NRC + NSUB - 1) // NSUB
F32 = jnp.float32
I32 = jnp.int32


def _pack_body(ns, nd, x_ref, w_ref, b_ref, *refs):
    m_refs = refs[:ns + nd]
    out_refs = refs[ns + nd:]
    c = pl.program_id(1)
    xh = jnp.dot(x_ref[...], w_ref[...], preferred_element_type=F32) + b_ref[...]
    xh_h = jnp.where(c == 0, xh[:, :16], xh[:, 16:])
    lanes = lax.broadcasted_iota(I32, (BN, 16), 1)
    for t in range(ns + nd):
        av = jnp.dot(xh, m_refs[t][...], preferred_element_type=F32)
        a2 = jnp.where(c == 0, av[:, :2], av[:, 2:])
        bc = jnp.where(lanes < 8, a2[:, 0:1], a2[:, 1:2])
        if t < ns:
            out_refs[t][...] = jnp.concatenate([xh_h, bc], axis=1)
        else:
            out_refs[t][...] = bc


def _pack_tables(x, w, b, src_ms, dst_ms):
    ns, nd = len(src_ms), len(dst_ms)
    din = x.shape[1]
    in_specs = [
        pl.BlockSpec((BN, din), lambda i, c: (i, 0)),
        pl.BlockSpec((din, HID), lambda i, c: (0, 0)),
        pl.BlockSpec((1, HID), lambda i, c: (0, 0)),
    ] + [pl.BlockSpec((HID, HEADS), lambda i, c: (0, 0))] * (ns + nd)
    out_specs = (
        [pl.BlockSpec((BN, 32), lambda i, c: (c * NB + i, 0))] * ns
        + [pl.BlockSpec((BN, 16), lambda i, c: (c * NB + i, 0))] * nd
    )
    out_shape = ([jax.ShapeDtypeStruct((2 * N, 32), F32)] * ns
                 + [jax.ShapeDtypeStruct((2 * N, 16), F32)] * nd)
    return pl.pallas_call(
        functools.partial(_pack_body, ns, nd),
        grid=(NB, 2),
        in_specs=in_specs,
        out_specs=out_specs,
        out_shape=out_shape,
    )(x, w, b[None, :], *src_ms, *dst_ms)


def _edge_body(src_hbm, dst_hbm, idxp_hbm, out_hbm,
               acc, ibuf, scidx, srcb, dstb, stage, zb, ob,
               si0, si1, si2, si3, sgs0, sgs1, sgs2, sgs3,
               sgd0, sgd1, sgd2, sgd3, ssc0, ssc1):
    c = lax.axis_index("c")
    s = lax.axis_index("s")
    si = (si0, si1, si2, si3)
    sgs = (sgs0, sgs1, sgs2, sgs3)
    sgd = (sgd0, sgd1, sgd2, sgd3)
    ssc = (ssc0, ssc1)
    ebase = s * EPT
    cnv = jnp.full((16,), c * N, I32)

    z16 = jnp.zeros((16,), F32)

    def zrow(r, _):
        zb[r, 0:16] = z16
        zb[r, 16:32] = z16
        return _

    lax.fori_loop(0, RC, zrow, None)

    def zcopy(t, _):
        cid = t * NSUB + s

        @pl.when(cid < NRC)
        def _():
            pltpu.sync_copy(zb, acc.at[pl.ds(cid * RC, RC)])

        return _

    lax.fori_loop(0, KMAX, zcopy, None)
    plsc.subcore_barrier()

    def issue_idx(k, b):
        eb = c * (2 * E) + (ebase + k * C) * 2
        pltpu.async_copy(idxp_hbm.at[pl.ds(eb, 2 * C)], ibuf.at[b], si[b])

    def wait_idx(b):
        pltpu.make_async_copy(idxp_hbm.at[pl.ds(0, 2 * C)], ibuf.at[b], si[b]).wait()

    def issue_gathers(b):
        pltpu.async_copy(src_hbm.at[ibuf.at[b, pl.ds(0, C)]], srcb.at[b], sgs[b])
        pltpu.async_copy(dst_hbm.at[ibuf.at[b, pl.ds(C, C)]], dstb.at[b], sgd[b])

    def wait_gathers(b):
        pltpu.make_async_copy(src_hbm.at[ibuf.at[b, pl.ds(0, C)]], srcb.at[b], sgs[b]).wait()
        pltpu.make_async_copy(dst_hbm.at[ibuf.at[b, pl.ds(C, C)]], dstb.at[b], sgd[b]).wait()

    def issue_scatter(b2):
        pltpu.async_copy(stage.at[b2], acc.at[scidx.at[b2]], ssc[b2], add=True)

    def wait_scatter(b2):
        pltpu.make_async_copy(stage.at[b2], acc.at[scidx.at[b2]], ssc[b2]).wait()

    def compute(b, b2):
        for i in range(C // 16):
            scidx[b2, pl.ds(i * 16, 16)] = ibuf[b, pl.ds(C + i * 16, 16)] - cnv

        def erow(j, _):
            sv0 = srcb[b, j, 0:16]
            sv1 = srcb[b, j, 16:32]
            dv = dstb[b, j, 0:16]
            sc8 = sv1 + dv
            e8 = jnp.exp(jnp.maximum(sc8, 0.2 * sc8))
            stage[b2, j, 0:16] = sv0 * e8
            stage[b2, j, 16:32] = e8
            return _

        lax.fori_loop(0, C, erow, None)

    LAST = NCH - 1

    def step(k, j, last=False):
        b = j % 4
        b2 = j % 2

        @pl.when(k >= 2)
        def _():
            wait_scatter(b2)

        wait_gathers(b)
        compute(b, b2)
        issue_scatter(b2)
        if not last:
            @pl.when(k + 4 <= LAST)
            def _():
                issue_idx(k + 4, b)

            @pl.when(k + 3 <= LAST)
            def _():
                wait_idx((j + 3) % 4)
                issue_gathers((j + 3) % 4)

    for k in range(4):
        issue_idx(k, k)
    for k in range(3):
        wait_idx(k)
        issue_gathers(k)

    def quad(q, _):
        for j in range(4):
            step(4 * q + j, j)
        return _

    lax.fori_loop(0, LAST // 4, quad, None)
    step(LAST, LAST % 4, last=True)
    wait_scatter(0)
    wait_scatter(1)
    plsc.subcore_barrier()

    def nchunk(t, _):
        cid = t * NSUB + s

        @pl.when(cid < NRC)
        def _():
            r0 = cid * RC
            pltpu.sync_copy(acc.at[pl.ds(r0, RC)], zb)

            def nrow(r, __):
                num = zb[r, 0:16]
                sden = zb[r, 16:32]
                ob[r, 0:16] = jnp.maximum(num / (sden + 1e-16), 0.0)
                return __

            lax.fori_loop(0, RC, nrow, None)
            pltpu.sync_copy(ob, out_hbm.at[c, pl.ds(r0, RC)])

        return _

    lax.fori_loop(0, KMAX, nchunk, None)


_EDGE_SCRATCH = [
    pltpu.VMEM_SHARED((N, 32), F32),
    pltpu.VMEM((4, 2 * C), I32),
    pltpu.VMEM((2, C), I32),
    pltpu.VMEM((4, C, 32), F32),
    pltpu.VMEM((4, C, 16), F32),
    pltpu.VMEM((2, C, 32), F32),
    pltpu.VMEM((RC, 32), F32),
    pltpu.VMEM((RC, 16), F32),
] + [pltpu.SemaphoreType.DMA] * 14


def _edge_conv(src_tab, dst_tab, idxp):
    k = pl.kernel(
        _edge_body,
        out_type=jax.ShapeDtypeStruct((2, N, 16), F32),
        mesh=plsc.VectorSubcoreMesh(core_axis_name="c", subcore_axis_name="s"),
        scratch_types=_EDGE_SCRATCH,
        compiler_params=pltpu.CompilerParams(use_tc_tiling_on_sc=False),
    )
    o = k(src_tab, dst_tab, idxp)
    return jnp.concatenate([o[0], o[1]], axis=1)


def _sem_stats_body(x0_ref, x1_ref, kw_ref, kb_ref, o_ref):
    i = pl.program_id(0)
    t0 = jnp.tanh(jnp.dot(x0_ref[...], kw_ref[...], preferred_element_type=F32) + kb_ref[...])
    t1 = jnp.tanh(jnp.dot(x1_ref[...], kw_ref[...], preferred_element_type=F32) + kb_ref[...])
    s0 = jnp.sum(t0, axis=0, keepdims=True)
    s1 = jnp.sum(t1, axis=0, keepdims=True)
    rows = lax.broadcasted_iota(I32, (8, HID), 0)
    contrib = jnp.where(rows == 0, s0, 0.0) + jnp.where(rows == 1, s1, 0.0)

    @pl.when(i == 0)
    def _():
        o_ref[...] = contrib

    @pl.when(i > 0)
    def _():
        o_ref[...] += contrib


def _sem_stats(x0, x1, kw, kb):
    return pl.pallas_call(
        _sem_stats_body,
        grid=(NB,),
        in_specs=[
            pl.BlockSpec((BN, HID), lambda i: (i, 0)),
            pl.BlockSpec((BN, HID), lambda i: (i, 0)),
            pl.BlockSpec((HID, HID), lambda i: (0, 0)),
            pl.BlockSpec((1, HID), lambda i: (0, 0)),
        ],
        out_specs=pl.BlockSpec((8, HID), lambda i: (0, 0)),
        out_shape=jax.ShapeDtypeStruct((8, HID), F32),
    )(x0, x1, kw, kb[None, :])


def _sem_weights(t_ref, q_ref):
    kv = t_ref[...] / N
    sc = jnp.sum(q_ref[...] * kv, axis=1)
    s0, s1 = sc[0], sc[1]
    m = jnp.maximum(s0, s1)
    e0 = jnp.exp(s0 - m)
    e1 = jnp.exp(s1 - m)
    return e0 / (e0 + e1), e1 / (e0 + e1)


def _sem_combine_body(t_ref, q_ref, x0_ref, x1_ref, o_ref):
    w0, w1 = _sem_weights(t_ref, q_ref)
    o_ref[...] = w0 * x0_ref[...] + w1 * x1_ref[...]


def _sem_combine_lin_body(t_ref, q_ref, x0_ref, x1_ref, lw_ref, lb_ref, o_ref):
    w0, w1 = _sem_weights(t_ref, q_ref)
    y = w0 * x0_ref[...] + w1 * x1_ref[...]
    o_ref[...] = jnp.dot(y, lw_ref[...], preferred_element_type=F32) + lb_ref[...]


def _sem_combine(t, q, x0, x1, lin=None):
    base_specs = [
        pl.BlockSpec((8, HID), lambda i: (0, 0)),
        pl.BlockSpec((1, HID), lambda i: (0, 0)),
        pl.BlockSpec((BN, HID), lambda i: (i, 0)),
        pl.BlockSpec((BN, HID), lambda i: (i, 0)),
    ]
    if lin is None:
        return pl.pallas_call(
            _sem_combine_body,
            grid=(NB,),
            in_specs=base_specs,
            out_specs=pl.BlockSpec((BN, HID), lambda i: (i, 0)),
            out_shape=jax.ShapeDtypeStruct((N, HID), F32),
        )(t, q[None, :], x0, x1)
    lw, lb = lin
    return pl.pallas_call(
        _sem_combine_lin_body,
        grid=(NB,),
        in_specs=base_specs + [
            pl.BlockSpec((HID, 8), lambda i: (0, 0)),
            pl.BlockSpec((1, 8), lambda i: (0, 0)),
        ],
        out_specs=pl.BlockSpec((BN, 8), lambda i: (i, 0)),
        out_shape=jax.ShapeDtypeStruct((N, 8), F32),
    )(t, q[None, :], x0, x1, lw, lb[None, :])


def _att_mat(att):
    return jnp.einsum("hd,hg->hdg", att, jnp.eye(HEADS, dtype=F32)).reshape(HID, HEADS)


def kernel(x_addr, x_tx, params, ei_a2t, ei_t2a, ei_a2a):
    rc = {}
    for name, ei in (("a2t", ei_a2t), ("t2a", ei_t2a), ("a2a", ei_a2a)):
        r2 = ei[0].reshape(-1, C)
        c2 = ei[1].reshape(-1, C)
        p0 = jnp.concatenate([r2, c2], axis=1).reshape(-1)
        p1 = jnp.concatenate([r2 + N, c2 + N], axis=1).reshape(-1)
        rc[name] = jnp.concatenate([p0, p1])

    x = {"addr": x_addr, "tx": x_tx}
    for l in (1, 2, 3):
        p = {k[len("c%d_" % l):]: v for k, v in params.items()
             if k.startswith("c%d_" % l)}
        et_a2t = "addr__to__tx"
        et_t2a = "tx__to__addr"
        et_a2a = "addr__self__addr"
        s_a2t, s_a2a, d_t2a, d_a2a = _pack_tables(
            x["addr"], p["proj_addr_w"], p["proj_addr_b"],
            [_att_mat(p["att_src_" + et_a2t]), _att_mat(p["att_src_" + et_a2a])],
            [_att_mat(p["att_dst_" + et_t2a]), _att_mat(p["att_dst_" + et_a2a])],
        )
        s_t2a, d_a2t = _pack_tables(
            x["tx"], p["proj_tx_w"], p["proj_tx_b"],
            [_att_mat(p["att_src_" + et_t2a])],
            [_att_mat(p["att_dst_" + et_a2t])],
        )
        o_a2t = _edge_conv(s_a2t, d_a2t, rc["a2t"])
        o_t2a = _edge_conv(s_t2a, d_t2a, rc["t2a"])
        o_a2a = _edge_conv(s_a2a, d_a2a, rc["a2a"])
        t_stats = _sem_stats(o_t2a, o_a2a, p["k_w"], p["k_b"])
        if l < 3:
            x = {"addr": _sem_combine(t_stats, p["q"], o_t2a, o_a2a),
                 "tx": o_a2t}
        else:
            lwp = jnp.zeros((HID, 8), F32).at[:, :2].set(params["lin_w"])
            lbp = jnp.zeros((8,), F32).at[:2].set(params["lin_b"])
            out = _sem_combine(t_stats, p["q"], o_t2a, o_a2a, lin=(lwp, lbp))
    return out[:, :2]


# speedup vs baseline: 1.5679x; 1.5679x over previous
import functools

import jax
import jax.numpy as jnp
from jax import lax
from jax.experimental import pallas as pl
from jax.experimental.pallas import tpu as pltpu
from jax.experimental.pallas import tpu_sc as plsc

N = 50000
E = 800000
HID = 32
HEADS = 4
DH = 8
BN = 400
NB = N // BN
C = 80
NSUB = 16
EPT = E // NSUB
NCH = EPT // C
RC = 200
NRC = N // RC
KMAX = (NRC + NSUB - 1) // NSUB
F32 = jnp.float32
I32 = jnp.int32


def _pack_body(ns, nd, x_ref, w_ref, b_ref, *refs):
    m_refs = refs[:ns + nd]
    out_refs = refs[ns + nd:]
    c = pl.program_id(1)
    xh = jnp.dot(x_ref[...], w_ref[...], preferred_element_type=F32) + b_ref[...]
    xh_h = jnp.where(c == 0, xh[:, :16], xh[:, 16:])
    lanes = lax.broadcasted_iota(I32, (BN, 16), 1)
    for t in range(ns + nd):
        av = jnp.dot(xh, m_refs[t][...], preferred_element_type=F32)
        a2 = jnp.where(c == 0, av[:, :2], av[:, 2:])
        bc = jnp.where(lanes < 8, a2[:, 0:1], a2[:, 1:2])
        if t < ns:
            out_refs[t][...] = jnp.concatenate([xh_h, bc], axis=1)
        else:
            out_refs[t][...] = bc


def _pack_tables(x, w, b, src_ms, dst_ms):
    ns, nd = len(src_ms), len(dst_ms)
    din = x.shape[1]
    in_specs = [
        pl.BlockSpec((BN, din), lambda i, c: (i, 0)),
        pl.BlockSpec((din, HID), lambda i, c: (0, 0)),
        pl.BlockSpec((1, HID), lambda i, c: (0, 0)),
    ] + [pl.BlockSpec((HID, HEADS), lambda i, c: (0, 0))] * (ns + nd)
    out_specs = (
        [pl.BlockSpec((BN, 32), lambda i, c: (c * NB + i, 0))] * ns
        + [pl.BlockSpec((BN, 16), lambda i, c: (c * NB + i, 0))] * nd
    )
    out_shape = ([jax.ShapeDtypeStruct((2 * N, 32), F32)] * ns
                 + [jax.ShapeDtypeStruct((2 * N, 16), F32)] * nd)
    return pl.pallas_call(
        functools.partial(_pack_body, ns, nd),
        grid=(NB, 2),
        in_specs=in_specs,
        out_specs=out_specs,
        out_shape=out_shape,
    )(x, w, b[None, :], *src_ms, *dst_ms)


def _edge_body(src_hbm, dst_hbm, idxp_hbm, out_hbm,
               acc, ibuf, scidx, srcb, dstb, stage, zb, ob,
               si0, si1, si2, si3, sgs0, sgs1, sgs2, sgs3,
               sgd0, sgd1, sgd2, sgd3, ssc0, ssc1):
    c = lax.axis_index("c")
    s = lax.axis_index("s")
    si = (si0, si1, si2, si3)
    sgs = (sgs0, sgs1, sgs2, sgs3)
    sgd = (sgd0, sgd1, sgd2, sgd3)
    ssc = (ssc0, ssc1)
    ebase = s * EPT
    cnv = jnp.full((16,), c * N, I32)

    z16 = jnp.zeros((16,), F32)

    def zrow(r, _):
        zb[r, 0:16] = z16
        zb[r, 16:32] = z16
        return _

    lax.fori_loop(0, RC, zrow, None)

    def zcopy(t, _):
        cid = t * NSUB + s

        @pl.when(cid < NRC)
        def _():
            pltpu.sync_copy(zb, acc.at[pl.ds(cid * RC, RC)])

        return _

    lax.fori_loop(0, KMAX, zcopy, None)
    plsc.subcore_barrier()

    def issue_idx(k, b):
        eb = c * (2 * E) + (ebase + k * C) * 2
        pltpu.async_copy(idxp_hbm.at[pl.ds(eb, 2 * C)], ibuf.at[b], si[b])

    def wait_idx(b):
        pltpu.make_async_copy(idxp_hbm.at[pl.ds(0, 2 * C)], ibuf.at[b], si[b]).wait()

    def issue_gathers(b):
        pltpu.async_copy(src_hbm.at[ibuf.at[b, pl.ds(0, C)]], srcb.at[b], sgs[b])
        pltpu.async_copy(dst_hbm.at[ibuf.at[b, pl.ds(C, C)]], dstb.at[b], sgd[b])

    def wait_gathers(b):
        pltpu.make_async_copy(src_hbm.at[ibuf.at[b, pl.ds(0, C)]], srcb.at[b], sgs[b]).wait()
        pltpu.make_async_copy(dst_hbm.at[ibuf.at[b, pl.ds(C, C)]], dstb.at[b], sgd[b]).wait()

    def issue_scatter(b2):
        pltpu.async_copy(stage.at[b2], acc.at[scidx.at[b2]], ssc[b2], add=True)

    def wait_scatter(b2):
        pltpu.make_async_copy(stage.at[b2], acc.at[scidx.at[b2]], ssc[b2]).wait()

    def compute(b, b2):
        for i in range(C // 16):
            scidx[b2, pl.ds(i * 16, 16)] = ibuf[b, pl.ds(C + i * 16, 16)] - cnv

        @plsc.parallel_loop(0, C, 1, unroll=4)
        def erow(j):
            sv0 = srcb[b, j, 0:16]
            sv1 = srcb[b, j, 16:32]
            dv = dstb[b, j, 0:16]
            sc8 = sv1 + dv
            e8 = jnp.exp(jnp.maximum(sc8, 0.2 * sc8))
            stage[b2, j, 0:16] = sv0 * e8
            stage[b2, j, 16:32] = e8

    LAST = NCH - 1

    def step(k, j, last=False):
        b = j % 4
        b2 = j % 2

        @pl.when(k >= 2)
        def _():
            wait_scatter(b2)

        wait_gathers(b)
        compute(b, b2)
        issue_scatter(b2)
        if not last:
            @pl.when(k + 4 <= LAST)
            def _():
                issue_idx(k + 4, b)

            @pl.when(k + 3 <= LAST)
            def _():
                wait_idx((j + 3) % 4)
                issue_gathers((j + 3) % 4)

    for k in range(4):
        issue_idx(k, k)
    for k in range(3):
        wait_idx(k)
        issue_gathers(k)

    def quad(q, _):
        for j in range(4):
            step(4 * q + j, j)
        return _

    lax.fori_loop(0, LAST // 4, quad, None)
    step(LAST, LAST % 4, last=True)
    wait_scatter(0)
    wait_scatter(1)
    plsc.subcore_barrier()

    def nchunk(t, _):
        cid = t * NSUB + s

        @pl.when(cid < NRC)
        def _():
            r0 = cid * RC
            pltpu.sync_copy(acc.at[pl.ds(r0, RC)], zb)

            def nrow(r, __):
                num = zb[r, 0:16]
                sden = zb[r, 16:32]
                ob[r, 0:16] = jnp.maximum(num / (sden + 1e-16), 0.0)
                return __

            lax.fori_loop(0, RC, nrow, None)
            pltpu.sync_copy(ob, out_hbm.at[c, pl.ds(r0, RC)])

        return _

    lax.fori_loop(0, KMAX, nchunk, None)


_EDGE_SCRATCH = [
    pltpu.VMEM_SHARED((N, 32), F32),
    pltpu.VMEM((4, 2 * C), I32),
    pltpu.VMEM((2, C), I32),
    pltpu.VMEM((4, C, 32), F32),
    pltpu.VMEM((4, C, 16), F32),
    pltpu.VMEM((2, C, 32), F32),
    pltpu.VMEM((RC, 32), F32),
    pltpu.VMEM((RC, 16), F32),
] + [pltpu.SemaphoreType.DMA] * 14


def _edge_conv(src_tab, dst_tab, idxp):
    k = pl.kernel(
        _edge_body,
        out_type=jax.ShapeDtypeStruct((2, N, 16), F32),
        mesh=plsc.VectorSubcoreMesh(core_axis_name="c", subcore_axis_name="s"),
        scratch_types=_EDGE_SCRATCH,
        compiler_params=pltpu.CompilerParams(use_tc_tiling_on_sc=False),
    )
    o = k(src_tab, dst_tab, idxp)
    return jnp.concatenate([o[0], o[1]], axis=1)


def _sem_stats_body(x0_ref, x1_ref, kw_ref, kb_ref, o_ref):
    i = pl.program_id(0)
    t0 = jnp.tanh(jnp.dot(x0_ref[...], kw_ref[...], preferred_element_type=F32) + kb_ref[...])
    t1 = jnp.tanh(jnp.dot(x1_ref[...], kw_ref[...], preferred_element_type=F32) + kb_ref[...])
    s0 = jnp.sum(t0, axis=0, keepdims=True)
    s1 = jnp.sum(t1, axis=0, keepdims=True)
    rows = lax.broadcasted_iota(I32, (8, HID), 0)
    contrib = jnp.where(rows == 0, s0, 0.0) + jnp.where(rows == 1, s1, 0.0)

    @pl.when(i == 0)
    def _():
        o_ref[...] = contrib

    @pl.when(i > 0)
    def _():
        o_ref[...] += contrib


def _sem_stats(x0, x1, kw, kb):
    return pl.pallas_call(
        _sem_stats_body,
        grid=(NB,),
        in_specs=[
            pl.BlockSpec((BN, HID), lambda i: (i, 0)),
            pl.BlockSpec((BN, HID), lambda i: (i, 0)),
            pl.BlockSpec((HID, HID), lambda i: (0, 0)),
            pl.BlockSpec((1, HID), lambda i: (0, 0)),
        ],
        out_specs=pl.BlockSpec((8, HID), lambda i: (0, 0)),
        out_shape=jax.ShapeDtypeStruct((8, HID), F32),
    )(x0, x1, kw, kb[None, :])


def _sem_weights(t_ref, q_ref):
    kv = t_ref[...] / N
    sc = jnp.sum(q_ref[...] * kv, axis=1)
    s0, s1 = sc[0], sc[1]
    m = jnp.maximum(s0, s1)
    e0 = jnp.exp(s0 - m)
    e1 = jnp.exp(s1 - m)
    return e0 / (e0 + e1), e1 / (e0 + e1)


def _sem_combine_body(t_ref, q_ref, x0_ref, x1_ref, o_ref):
    w0, w1 = _sem_weights(t_ref, q_ref)
    o_ref[...] = w0 * x0_ref[...] + w1 * x1_ref[...]


def _sem_combine_lin_body(t_ref, q_ref, x0_ref, x1_ref, lw_ref, lb_ref, o_ref):
    w0, w1 = _sem_weights(t_ref, q_ref)
    y = w0 * x0_ref[...] + w1 * x1_ref[...]
    o_ref[...] = jnp.dot(y, lw_ref[...], preferred_element_type=F32) + lb_ref[...]


def _sem_combine(t, q, x0, x1, lin=None):
    base_specs = [
        pl.BlockSpec((8, HID), lambda i: (0, 0)),
        pl.BlockSpec((1, HID), lambda i: (0, 0)),
        pl.BlockSpec((BN, HID), lambda i: (i, 0)),
        pl.BlockSpec((BN, HID), lambda i: (i, 0)),
    ]
    if lin is None:
        return pl.pallas_call(
            _sem_combine_body,
            grid=(NB,),
            in_specs=base_specs,
            out_specs=pl.BlockSpec((BN, HID), lambda i: (i, 0)),
            out_shape=jax.ShapeDtypeStruct((N, HID), F32),
        )(t, q[None, :], x0, x1)
    lw, lb = lin
    return pl.pallas_call(
        _sem_combine_lin_body,
        grid=(NB,),
        in_specs=base_specs + [
            pl.BlockSpec((HID, 8), lambda i: (0, 0)),
            pl.BlockSpec((1, 8), lambda i: (0, 0)),
        ],
        out_specs=pl.BlockSpec((BN, 8), lambda i: (i, 0)),
        out_shape=jax.ShapeDtypeStruct((N, 8), F32),
    )(t, q[None, :], x0, x1, lw, lb[None, :])


def _att_mat(att):
    return jnp.einsum("hd,hg->hdg", att, jnp.eye(HEADS, dtype=F32)).reshape(HID, HEADS)


def kernel(x_addr, x_tx, params, ei_a2t, ei_t2a, ei_a2a):
    rc = {}
    for name, ei in (("a2t", ei_a2t), ("t2a", ei_t2a), ("a2a", ei_a2a)):
        r2 = ei[0].reshape(-1, C)
        c2 = ei[1].reshape(-1, C)
        p0 = jnp.concatenate([r2, c2], axis=1).reshape(-1)
        p1 = jnp.concatenate([r2 + N, c2 + N], axis=1).reshape(-1)
        rc[name] = jnp.concatenate([p0, p1])

    x = {"addr": x_addr, "tx": x_tx}
    for l in (1, 2, 3):
        p = {k[len("c%d_" % l):]: v for k, v in params.items()
             if k.startswith("c%d_" % l)}
        et_a2t = "addr__to__tx"
        et_t2a = "tx__to__addr"
        et_a2a = "addr__self__addr"
        s_a2t, s_a2a, d_t2a, d_a2a = _pack_tables(
            x["addr"], p["proj_addr_w"], p["proj_addr_b"],
            [_att_mat(p["att_src_" + et_a2t]), _att_mat(p["att_src_" + et_a2a])],
            [_att_mat(p["att_dst_" + et_t2a]), _att_mat(p["att_dst_" + et_a2a])],
        )
        s_t2a, d_a2t = _pack_tables(
            x["tx"], p["proj_tx_w"], p["proj_tx_b"],
            [_att_mat(p["att_src_" + et_t2a])],
            [_att_mat(p["att_dst_" + et_a2t])],
        )
        o_a2t = _edge_conv(s_a2t, d_a2t, rc["a2t"])
        o_t2a = _edge_conv(s_t2a, d_t2a, rc["t2a"])
        o_a2a = _edge_conv(s_a2a, d_a2a, rc["a2a"])
        t_stats = _sem_stats(o_t2a, o_a2a, p["k_w"], p["k_b"])
        if l < 3:
            x = {"addr": _sem_combine(t_stats, p["q"], o_t2a, o_a2a),
                 "tx": o_a2t}
        else:
            lwp = jnp.zeros((HID, 8), F32).at[:, :2].set(params["lin_w"])
            lbp = jnp.zeros((8,), F32).at[:2].set(params["lin_b"])
            out = _sem_combine(t_stats, p["q"], o_t2a, o_a2a, lin=(lwp, lbp))
    return out[:, :2]
